# baseline (speedup 1.0000x reference)
import functools

import jax
import jax.numpy as jnp
from jax import lax
from jax.experimental import pallas as pl
from jax.experimental.pallas import tpu as pltpu
from jax.experimental.pallas import tpu_sc as plsc

N_E = 1024
E_DIM = 256
N_TOK = 16384
TOK_BLK = 1024

_SC_CORES = 2
_SC_SUBCORES = 16
_NW = _SC_CORES * _SC_SUBCORES
_CHUNK = 128

_ROWS_PER_VREG = 8


def _dist_argmin_body(n_blk, z_ref, cb_ref, idx_ref, loss_ref, b_ref):
    i = pl.program_id(0)
    zblk = z_ref[...]
    cb = cb_ref[...]
    cb2 = cb + cb
    m2 = lax.dot_general(cb2, zblk, (((1,), (1,)), ((), ())),
                         preferred_element_type=jnp.float32)
    zsq = zblk * zblk
    ones_row = jnp.ones((1, E_DIM), jnp.float32)
    a = lax.dot_general(ones_row, zsq, (((1,), (1,)), ((), ())),
                        preferred_element_type=jnp.float32)

    @pl.when(i == 0)
    def _precompute_b():
        b_ref[...] = jnp.sum(cb * cb, axis=1)[:, None]

    b = b_ref[...]
    d = (a + b) - m2
    ds3 = d.reshape(N_E // _ROWS_PER_VREG, _ROWS_PER_VREG, TOK_BLK)
    mv = ds3[0]
    mi = jnp.zeros((_ROWS_PER_VREG, TOK_BLK), jnp.int32)
    for r in range(1, N_E // _ROWS_PER_VREG):
        row = ds3[r]
        lt = row < mv
        mv = jnp.where(lt, row, mv)
        mi = jnp.where(lt, r, mi)
    srow = lax.broadcasted_iota(jnp.int32, (_ROWS_PER_VREG, TOK_BLK), 0)
    gi = mi * _ROWS_PER_VREG + srow
    mind = jnp.min(mv, axis=0, keepdims=True)
    idx = jnp.min(jnp.where(mv == mind, gi, N_E), axis=0)
    idx_ref[0, 0, :] = idx

    @pl.when(i == 0)
    def _init():
        loss_ref[0, 0] = 0.0

    loss_ref[0, 0] += jnp.sum(mind)

    @pl.when(i == n_blk - 1)
    def _finish():
        loss_ref[0, 0] = loss_ref[0, 0] * (1.25 / (N_TOK * E_DIM))


def _dist_argmin(z_flat, codebook):
    n_tok = z_flat.shape[0]
    n_blk = n_tok // TOK_BLK
    return pl.pallas_call(
        functools.partial(_dist_argmin_body, n_blk),
        grid=(n_blk,),
        in_specs=[
            pl.BlockSpec((TOK_BLK, E_DIM), lambda i: (i, 0)),
            pl.BlockSpec((N_E, E_DIM), lambda i: (0, 0)),
        ],
        out_specs=[
            pl.BlockSpec((1, 1, TOK_BLK), lambda i: (i, 0, 0)),
            pl.BlockSpec(block_shape=(1, 1), index_map=lambda i: (0, 0),
                         memory_space=pltpu.SMEM),
        ],
        out_shape=[
            jax.ShapeDtypeStruct((n_blk, 1, TOK_BLK), jnp.int32),
            jax.ShapeDtypeStruct((1, 1), jnp.float32),
        ],
        scratch_shapes=[pltpu.VMEM((N_E, 1), jnp.float32)],
    )(z_flat, codebook)


@functools.cache
def _gather_fn(n_tok):
    mesh = plsc.VectorSubcoreMesh(core_axis_name="c", subcore_axis_name="s")
    b_per_w = n_tok // _NW
    n_chunk = b_per_w // _CHUNK

    @functools.partial(
        pl.kernel, mesh=mesh,
        out_type=jax.ShapeDtypeStruct((n_tok, E_DIM), jnp.float32),
        scratch_types=[
            pltpu.VMEM((b_per_w,), jnp.int32),
            pltpu.VMEM((_CHUNK, E_DIM), jnp.float32),
            pltpu.VMEM((_CHUNK, E_DIM), jnp.float32),
            pltpu.SemaphoreType.DMA,
            pltpu.SemaphoreType.DMA,
        ],
    )
    def gather(cb_hbm, idx_hbm, out_hbm, idx_v, buf0, buf1, sem0, sem1):
        wid = lax.axis_index("s") * _SC_CORES + lax.axis_index("c")
        base = wid * b_per_w
        pltpu.sync_copy(idx_hbm.at[pl.ds(base, b_per_w)], idx_v)
        bufs = (buf0, buf1)
        sems = (sem0, sem1)
        copies = []
        for k in range(n_chunk):
            copies.append(pltpu.async_copy(
                cb_hbm.at[idx_v.at[pl.ds(k * _CHUNK, _CHUNK)]],
                bufs[k % 2], sems[k % 2]))
            if k >= 1:
                copies[k - 1].wait()
                pltpu.sync_copy(bufs[(k - 1) % 2],
                                out_hbm.at[pl.ds(base + (k - 1) * _CHUNK, _CHUNK)])
        copies[-1].wait()
        pltpu.sync_copy(bufs[(n_chunk - 1) % 2],
                        out_hbm.at[pl.ds(base + (n_chunk - 1) * _CHUNK, _CHUNK)])

    return gather


def kernel(z, codebook):
    z_flat = jnp.transpose(z, (0, 2, 3, 1)).reshape(N_TOK, E_DIM)
    half = N_TOK // 2
    idx3a, loss_a = _dist_argmin(z_flat[:half], codebook)
    zq_a = _gather_fn(half)(codebook, idx3a.reshape(half))
    idx3b, loss_b = _dist_argmin(z_flat[half:], codebook)
    zq_b = _gather_fn(half)(codebook, idx3b.reshape(half))
    zq_flat = jnp.concatenate([zq_a, zq_b], axis=0)
    idx = jnp.concatenate([idx3a.reshape(half), idx3b.reshape(half)])
    z_q_out = jnp.transpose(zq_flat.reshape(16, 32, 32, E_DIM), (0, 3, 1, 2))
    return z_q_out, loss_a[0, 0] + loss_b[0, 0], idx


# speedup vs baseline: 1.4137x; 1.4137x over previous
import functools

import jax
import jax.numpy as jnp
from jax import lax
from jax.experimental import pallas as pl
from jax.experimental.pallas import tpu as pltpu
from jax.experimental.pallas import tpu_sc as plsc

N_E = 1024
E_DIM = 256
N_TOK = 16384
TOK_BLK = 1024

_SC_CORES = 2
_SC_SUBCORES = 16
_NW = _SC_CORES * _SC_SUBCORES
_CHUNK = 128

_ROWS_PER_VREG = 8


def _dist_argmin_body(n_blk, z_ref, cb_ref, idx_ref, loss_ref, b_ref):
    i = pl.program_id(0)
    zblk = z_ref[...]
    cb = cb_ref[...]
    cb2 = cb + cb
    m2 = lax.dot_general(cb2, zblk, (((1,), (1,)), ((), ())),
                         preferred_element_type=jnp.float32)
    zsq = zblk * zblk
    ones_row = jnp.ones((1, E_DIM), jnp.float32)
    a = lax.dot_general(ones_row, zsq, (((1,), (1,)), ((), ())),
                        preferred_element_type=jnp.float32)

    @pl.when(i == 0)
    def _precompute_b():
        b_ref[...] = jnp.sum(cb * cb, axis=1)[:, None]

    b = b_ref[...]
    d = (a + b) - m2
    ds3 = d.reshape(N_E // _ROWS_PER_VREG, _ROWS_PER_VREG, TOK_BLK)
    mv = ds3[0]
    mi = jnp.zeros((_ROWS_PER_VREG, TOK_BLK), jnp.int32)
    for r in range(1, N_E // _ROWS_PER_VREG):
        row = ds3[r]
        lt = row < mv
        mv = jnp.where(lt, row, mv)
        mi = jnp.where(lt, r, mi)
    srow = lax.broadcasted_iota(jnp.int32, (_ROWS_PER_VREG, TOK_BLK), 0)
    gi = mi * _ROWS_PER_VREG + srow
    mind = jnp.min(mv, axis=0, keepdims=True)
    idx = jnp.min(jnp.where(mv == mind, gi, N_E), axis=0)
    idx_ref[0, 0, :] = idx

    @pl.when(i == 0)
    def _init():
        loss_ref[0, 0] = 0.0

    loss_ref[0, 0] += jnp.sum(mind)

    @pl.when(i == n_blk - 1)
    def _finish():
        loss_ref[0, 0] = loss_ref[0, 0] * (1.25 / (N_TOK * E_DIM))


def _dist_argmin(z_flat, codebook):
    n_tok = z_flat.shape[0]
    n_blk = n_tok // TOK_BLK
    return pl.pallas_call(
        functools.partial(_dist_argmin_body, n_blk),
        grid=(n_blk,),
        in_specs=[
            pl.BlockSpec((TOK_BLK, E_DIM), lambda i: (i, 0)),
            pl.BlockSpec((N_E, E_DIM), lambda i: (0, 0)),
        ],
        out_specs=[
            pl.BlockSpec((1, 1, TOK_BLK), lambda i: (i, 0, 0)),
            pl.BlockSpec(block_shape=(1, 1), index_map=lambda i: (0, 0),
                         memory_space=pltpu.SMEM),
        ],
        out_shape=[
            jax.ShapeDtypeStruct((n_blk, 1, TOK_BLK), jnp.int32),
            jax.ShapeDtypeStruct((1, 1), jnp.float32),
        ],
        scratch_shapes=[pltpu.VMEM((N_E, 1), jnp.float32)],
    )(z_flat, codebook)


@functools.cache
def _gather_fn(n_tok):
    mesh = plsc.VectorSubcoreMesh(core_axis_name="c", subcore_axis_name="s")
    b_per_w = n_tok // _NW
    n_chunk = b_per_w // _CHUNK

    @functools.partial(
        pl.kernel, mesh=mesh,
        out_type=jax.ShapeDtypeStruct((n_tok, E_DIM), jnp.float32),
        scratch_types=[
            pltpu.VMEM((b_per_w,), jnp.int32),
            pltpu.VMEM((_CHUNK, E_DIM), jnp.float32),
            pltpu.VMEM((_CHUNK, E_DIM), jnp.float32),
            pltpu.SemaphoreType.DMA,
            pltpu.SemaphoreType.DMA,
        ],
    )
    def gather(cb_hbm, idx_hbm, out_hbm, idx_v, buf0, buf1, sem0, sem1):
        wid = lax.axis_index("s") * _SC_CORES + lax.axis_index("c")
        base = wid * b_per_w
        pltpu.sync_copy(idx_hbm.at[pl.ds(base, b_per_w)], idx_v)
        bufs = (buf0, buf1)
        sems = (sem0, sem1)
        copies = []
        for k in range(n_chunk):
            copies.append(pltpu.async_copy(
                cb_hbm.at[idx_v.at[pl.ds(k * _CHUNK, _CHUNK)]],
                bufs[k % 2], sems[k % 2]))
            if k >= 1:
                copies[k - 1].wait()
                pltpu.sync_copy(bufs[(k - 1) % 2],
                                out_hbm.at[pl.ds(base + (k - 1) * _CHUNK, _CHUNK)])
        copies[-1].wait()
        pltpu.sync_copy(bufs[(n_chunk - 1) % 2],
                        out_hbm.at[pl.ds(base + (n_chunk - 1) * _CHUNK, _CHUNK)])

    return gather


def kernel(z, codebook):
    z_flat = jnp.transpose(z, (0, 2, 3, 1)).reshape(N_TOK, E_DIM)
    idx3, loss11 = _dist_argmin(z_flat, codebook)
    idx = idx3.reshape(N_TOK)
    zq_flat = _gather_fn(N_TOK)(codebook, idx)
    z_q_out = jnp.transpose(zq_flat.reshape(16, 32, 32, E_DIM), (0, 3, 1, 2))
    return z_q_out, loss11[0, 0], idx


# speedup vs baseline: 1.4253x; 1.0082x over previous
import functools

import jax
import jax.numpy as jnp
from jax import lax
from jax.experimental import pallas as pl
from jax.experimental.pallas import tpu as pltpu
from jax.experimental.pallas import tpu_sc as plsc

N_E = 1024
E_DIM = 256
N_TOK = 16384
TOK_BLK = 2048

_SC_CORES = 2
_SC_SUBCORES = 16
_NW = _SC_CORES * _SC_SUBCORES
_CHUNK = 128

_ROWS_PER_VREG = 8


def _dist_argmin_body(n_blk, z_ref, cb_ref, idx_ref, loss_ref, b_ref):
    i = pl.program_id(0)
    zblk = z_ref[...]
    cb = cb_ref[...]
    cb2 = cb + cb
    m2 = lax.dot_general(cb2, zblk, (((1,), (1,)), ((), ())),
                         preferred_element_type=jnp.float32)
    zsq = zblk * zblk
    ones_row = jnp.ones((1, E_DIM), jnp.float32)
    a = lax.dot_general(ones_row, zsq, (((1,), (1,)), ((), ())),
                        preferred_element_type=jnp.float32)

    @pl.when(i == 0)
    def _precompute_b():
        b_ref[...] = jnp.sum(cb * cb, axis=1)[:, None]

    b = b_ref[...]
    d = (a + b) - m2
    ds3 = d.reshape(N_E // _ROWS_PER_VREG, _ROWS_PER_VREG, TOK_BLK)
    mv = ds3[0]
    mi = jnp.zeros((_ROWS_PER_VREG, TOK_BLK), jnp.int32)
    for r in range(1, N_E // _ROWS_PER_VREG):
        row = ds3[r]
        lt = row < mv
        mv = jnp.where(lt, row, mv)
        mi = jnp.where(lt, r, mi)
    srow = lax.broadcasted_iota(jnp.int32, (_ROWS_PER_VREG, TOK_BLK), 0)
    gi = mi * _ROWS_PER_VREG + srow
    mind = jnp.min(mv, axis=0, keepdims=True)
    idx = jnp.min(jnp.where(mv == mind, gi, N_E), axis=0)
    idx_ref[0, 0, :] = idx

    @pl.when(i == 0)
    def _init():
        loss_ref[0, 0] = 0.0

    loss_ref[0, 0] += jnp.sum(mind)

    @pl.when(i == n_blk - 1)
    def _finish():
        loss_ref[0, 0] = loss_ref[0, 0] * (1.25 / (N_TOK * E_DIM))


def _dist_argmin(z_flat, codebook):
    n_tok = z_flat.shape[0]
    n_blk = n_tok // TOK_BLK
    return pl.pallas_call(
        functools.partial(_dist_argmin_body, n_blk),
        grid=(n_blk,),
        in_specs=[
            pl.BlockSpec((TOK_BLK, E_DIM), lambda i: (i, 0)),
            pl.BlockSpec((N_E, E_DIM), lambda i: (0, 0)),
        ],
        out_specs=[
            pl.BlockSpec((1, 1, TOK_BLK), lambda i: (i, 0, 0)),
            pl.BlockSpec(block_shape=(1, 1), index_map=lambda i: (0, 0),
                         memory_space=pltpu.SMEM),
        ],
        out_shape=[
            jax.ShapeDtypeStruct((n_blk, 1, TOK_BLK), jnp.int32),
            jax.ShapeDtypeStruct((1, 1), jnp.float32),
        ],
        scratch_shapes=[pltpu.VMEM((N_E, 1), jnp.float32)],
    )(z_flat, codebook)


@functools.cache
def _gather_fn(n_tok):
    mesh = plsc.VectorSubcoreMesh(core_axis_name="c", subcore_axis_name="s")
    b_per_w = n_tok // _NW
    n_chunk = b_per_w // _CHUNK

    @functools.partial(
        pl.kernel, mesh=mesh,
        out_type=jax.ShapeDtypeStruct((n_tok, E_DIM), jnp.float32),
        scratch_types=[
            pltpu.VMEM((b_per_w,), jnp.int32),
            pltpu.VMEM((_CHUNK, E_DIM), jnp.float32),
            pltpu.VMEM((_CHUNK, E_DIM), jnp.float32),
            pltpu.SemaphoreType.DMA,
            pltpu.SemaphoreType.DMA,
        ],
    )
    def gather(cb_hbm, idx_hbm, out_hbm, idx_v, buf0, buf1, sem0, sem1):
        wid = lax.axis_index("s") * _SC_CORES + lax.axis_index("c")
        base = wid * b_per_w
        pltpu.sync_copy(idx_hbm.at[pl.ds(base, b_per_w)], idx_v)
        bufs = (buf0, buf1)
        sems = (sem0, sem1)
        copies = []
        for k in range(n_chunk):
            copies.append(pltpu.async_copy(
                cb_hbm.at[idx_v.at[pl.ds(k * _CHUNK, _CHUNK)]],
                bufs[k % 2], sems[k % 2]))
            if k >= 1:
                copies[k - 1].wait()
                pltpu.sync_copy(bufs[(k - 1) % 2],
                                out_hbm.at[pl.ds(base + (k - 1) * _CHUNK, _CHUNK)])
        copies[-1].wait()
        pltpu.sync_copy(bufs[(n_chunk - 1) % 2],
                        out_hbm.at[pl.ds(base + (n_chunk - 1) * _CHUNK, _CHUNK)])

    return gather


def kernel(z, codebook):
    z_flat = jnp.transpose(z, (0, 2, 3, 1)).reshape(N_TOK, E_DIM)
    idx3, loss11 = _dist_argmin(z_flat, codebook)
    idx = idx3.reshape(N_TOK)
    zq_flat = _gather_fn(N_TOK)(codebook, idx)
    z_q_out = jnp.transpose(zq_flat.reshape(16, 32, 32, E_DIM), (0, 3, 1, 2))
    return z_q_out, loss11[0, 0], idx
